# speedup vs baseline: 17.6582x; 17.6582x over previous
import functools

import jax
import jax.numpy as jnp
from jax import lax
from jax.experimental import pallas as pl
from jax.experimental.pallas import tpu as pltpu
from jax.experimental.pallas import tpu_sc as plsc

_CHUNK = 80


def _sc_dims():
    try:
        info = plsc.get_sparse_core_info()
        return info.num_cores, info.num_subcores
    except Exception:
        return 2, 16


def _fill_1d(ref, n, value):
    v = jnp.full((16,), value, jnp.float32)

    def body(j, c):
        ref[pl.ds(j * 16, 16)] = v
        return c

    lax.fori_loop(0, n // 16, body, 0)


def _zero_2d(ref, rows, cols):
    z = jnp.zeros((16,), jnp.float32)
    per_row = cols // 16

    def body(j, c):
        ref[j // per_row, pl.ds((j % per_row) * 16, 16)] = z
        return c

    lax.fori_loop(0, rows * per_row, body, 0)


def _make_degree_kernel(E, n_pad):
    NC, NS = _sc_dims()
    e_per_w = E // (NC * NS)
    n_chunks = e_per_w // _CHUNK
    stripe = n_pad // NS
    mesh = plsc.VectorSubcoreMesh(core_axis_name="c", subcore_axis_name="s")

    @functools.partial(
        pl.kernel,
        out_type=jax.ShapeDtypeStruct((NC, n_pad), jnp.float32),
        mesh=mesh,
        scratch_types=[
            pltpu.VMEM((_CHUNK,), jnp.int32),
            pltpu.VMEM((_CHUNK,), jnp.float32),
            pltpu.VMEM((stripe,), jnp.float32),
            pltpu.VMEM_SHARED((n_pad,), jnp.float32),
        ],
    )
    def deg_kernel(dst_hbm, out_hbm, idx_v, ones_v, zbuf_v, deg_sh):
        cid = lax.axis_index("c")
        sid = lax.axis_index("s")
        base = (cid * NS + sid) * e_per_w
        _fill_1d(zbuf_v, stripe, 0.0)
        _fill_1d(ones_v, _CHUNK, 1.0)
        pltpu.sync_copy(zbuf_v, deg_sh.at[pl.ds(sid * stripe, stripe)])
        plsc.subcore_barrier()

        def body(i, c):
            pltpu.sync_copy(dst_hbm.at[pl.ds(base + i * _CHUNK, _CHUNK)], idx_v)
            pltpu.sync_copy(ones_v, deg_sh.at[idx_v], add=True)
            return c

        lax.fori_loop(0, n_chunks, body, 0)
        plsc.subcore_barrier()
        pltpu.sync_copy(
            deg_sh.at[pl.ds(sid * stripe, stripe)],
            out_hbm.at[cid, pl.ds(sid * stripe, stripe)],
        )

    return deg_kernel


def _make_msg_kernel(E, n_pad, D):
    NC, NS = _sc_dims()
    e_per_w = E // (NC * NS)
    n_chunks = e_per_w // _CHUNK
    stripe = n_pad // NS
    mesh = plsc.VectorSubcoreMesh(core_axis_name="c", subcore_axis_name="s")

    @functools.partial(
        pl.kernel,
        out_type=jax.ShapeDtypeStruct((NC, n_pad, D), jnp.float32),
        mesh=mesh,
        scratch_types=[
            pltpu.VMEM((_CHUNK,), jnp.int32),
            pltpu.VMEM((_CHUNK,), jnp.int32),
            pltpu.VMEM((_CHUNK, D), jnp.float32),
            pltpu.VMEM_SHARED((n_pad, D), jnp.float32),
            pltpu.SemaphoreType.DMA,
        ],
    )
    def msg_kernel(xs_hbm, src_hbm, dst_hbm, out_hbm, sidx_v, didx_v, rows_v, acc_sh, sem):
        cid = lax.axis_index("c")
        sid = lax.axis_index("s")
        base = (cid * NS + sid) * e_per_w
        row0 = sid * stripe
        _zero_2d(rows_v, _CHUNK, D)

        def zbody(r, c):
            pltpu.sync_copy(rows_v, acc_sh.at[pl.ds(row0 + r * _CHUNK, _CHUNK)])
            return c

        lax.fori_loop(0, stripe // _CHUNK, zbody, 0)
        plsc.subcore_barrier()

        def body(i, c):
            e0 = base + i * _CHUNK
            pltpu.sync_copy(src_hbm.at[pl.ds(e0, _CHUNK)], sidx_v)
            pltpu.sync_copy(dst_hbm.at[pl.ds(e0, _CHUNK)], didx_v)
            pltpu.async_copy(xs_hbm.at[sidx_v], rows_v, sem).wait()
            pltpu.sync_copy(rows_v, acc_sh.at[didx_v], add=True)
            return c

        lax.fori_loop(0, n_chunks, body, 0)
        plsc.subcore_barrier()
        pltpu.sync_copy(
            acc_sh.at[pl.ds(row0, stripe)],
            out_hbm.at[cid, pl.ds(row0, stripe)],
        )

    return msg_kernel


def _scale_body(deg_ref, x_ref, xs_ref, dinv_ref):
    dinv = lax.rsqrt(deg_ref[0] + deg_ref[1] + 1.0)
    xs_ref[...] = x_ref[...] * dinv
    dinv_ref[...] = dinv


def _make_scale_kernel(N, D, block_rows):
    return pl.pallas_call(
        _scale_body,
        grid=(N // block_rows,),
        in_specs=[
            pl.BlockSpec((2, block_rows, 1), lambda i: (0, i, 0)),
            pl.BlockSpec((block_rows, D), lambda i: (i, 0)),
        ],
        out_specs=[
            pl.BlockSpec((block_rows, D), lambda i: (i, 0)),
            pl.BlockSpec((block_rows, 1), lambda i: (i, 0)),
        ],
        out_shape=[
            jax.ShapeDtypeStruct((N, D), jnp.float32),
            jax.ShapeDtypeStruct((N, 1), jnp.float32),
        ],
    )


def _final_body(acc_ref, xs_ref, dinv_ref, w_ref, b_ref, out_ref):
    h = (acc_ref[0] + acc_ref[1] + xs_ref[...]) * dinv_ref[...]
    out_ref[...] = (
        jnp.dot(h, w_ref[...], preferred_element_type=jnp.float32) + b_ref[...]
    )


def _make_final_kernel(N, D, block_rows):
    return pl.pallas_call(
        _final_body,
        grid=(N // block_rows,),
        in_specs=[
            pl.BlockSpec((2, block_rows, D), lambda i: (0, i, 0)),
            pl.BlockSpec((block_rows, D), lambda i: (i, 0)),
            pl.BlockSpec((block_rows, 1), lambda i: (i, 0)),
            pl.BlockSpec((D, D), lambda i: (0, 0)),
            pl.BlockSpec((1, D), lambda i: (0, 0)),
        ],
        out_specs=pl.BlockSpec((block_rows, D), lambda i: (i, 0)),
        out_shape=jax.ShapeDtypeStruct((N, D), jnp.float32),
    )


def kernel(x, edge_index, W, b):
    N, D = x.shape
    E = edge_index.shape[1]
    NC, NS = _sc_dims()
    unit = NS * _CHUNK
    n_pad = ((N + unit - 1) // unit) * unit

    ei = edge_index.astype(jnp.int32)
    src = ei[0]
    dst = ei[1]

    deg_part = _make_degree_kernel(E, n_pad)(dst)
    deg3 = deg_part[:, :N].reshape(NC, N, 1)

    block_rows = 1000 if N % 1000 == 0 else 8
    xs, dinv = _make_scale_kernel(N, D, block_rows)(deg3, x)

    acc_part = _make_msg_kernel(E, n_pad, D)(xs, src, dst)
    acc = acc_part[:, :N, :]

    out = _make_final_kernel(N, D, block_rows)(acc, xs, dinv, W, b.reshape(1, D))
    return out


# speedup vs baseline: 24.6882x; 1.3981x over previous
import functools

import jax
import jax.numpy as jnp
from jax import lax
from jax.experimental import pallas as pl
from jax.experimental.pallas import tpu as pltpu
from jax.experimental.pallas import tpu_sc as plsc

_CHUNK = 80


def _sc_dims():
    try:
        info = plsc.get_sparse_core_info()
        return info.num_cores, info.num_subcores
    except Exception:
        return 2, 16


def _fill_1d(ref, n, value):
    v = jnp.full((16,), value, jnp.float32)

    def body(j, c):
        ref[pl.ds(j * 16, 16)] = v
        return c

    lax.fori_loop(0, n // 16, body, 0)


def _zero_2d(ref, rows, cols):
    z = jnp.zeros((16,), jnp.float32)
    per_row = cols // 16

    def body(j, c):
        ref[j // per_row, pl.ds((j % per_row) * 16, 16)] = z
        return c

    lax.fori_loop(0, rows * per_row, body, 0)


def _make_degree_kernel(E, n_pad):
    NC, NS = _sc_dims()
    e_per_w = E // (NC * NS)
    n_chunks = e_per_w // _CHUNK
    stripe = n_pad // NS
    mesh = plsc.VectorSubcoreMesh(core_axis_name="c", subcore_axis_name="s")

    @functools.partial(
        pl.kernel,
        out_type=jax.ShapeDtypeStruct((NC, n_pad), jnp.float32),
        mesh=mesh,
        scratch_types=[
            pltpu.VMEM((_CHUNK,), jnp.int32),
            pltpu.VMEM((_CHUNK,), jnp.float32),
            pltpu.VMEM((stripe,), jnp.float32),
            pltpu.VMEM_SHARED((n_pad,), jnp.float32),
        ],
    )
    def deg_kernel(dst_hbm, out_hbm, idx_v, ones_v, zbuf_v, deg_sh):
        cid = lax.axis_index("c")
        sid = lax.axis_index("s")
        base = (cid * NS + sid) * e_per_w
        _fill_1d(zbuf_v, stripe, 0.0)
        _fill_1d(ones_v, _CHUNK, 1.0)
        pltpu.sync_copy(zbuf_v, deg_sh.at[pl.ds(sid * stripe, stripe)])
        plsc.subcore_barrier()

        def body(i, c):
            pltpu.sync_copy(dst_hbm.at[pl.ds(base + i * _CHUNK, _CHUNK)], idx_v)
            pltpu.sync_copy(ones_v, deg_sh.at[idx_v], add=True)
            return c

        lax.fori_loop(0, n_chunks, body, 0)
        plsc.subcore_barrier()
        pltpu.sync_copy(
            deg_sh.at[pl.ds(sid * stripe, stripe)],
            out_hbm.at[cid, pl.ds(sid * stripe, stripe)],
        )

    return deg_kernel


def _make_msg_kernel(E, n_pad, D):
    NC, NS = _sc_dims()
    e_per_w = E // (NC * NS)
    n_chunks = e_per_w // _CHUNK
    stripe = n_pad // NS
    mesh = plsc.VectorSubcoreMesh(core_axis_name="c", subcore_axis_name="s")

    @functools.partial(
        pl.kernel,
        out_type=jax.ShapeDtypeStruct((NC, n_pad, D), jnp.float32),
        mesh=mesh,
        scratch_types=[
            pltpu.VMEM((2, _CHUNK), jnp.int32),
            pltpu.VMEM((2, _CHUNK), jnp.int32),
            pltpu.VMEM((2, _CHUNK, D), jnp.float32),
            pltpu.VMEM_SHARED((n_pad, D), jnp.float32),
            pltpu.SemaphoreType.DMA,
        ],
    )
    def msg_kernel(xs_hbm, src_hbm, dst_hbm, out_hbm, sidx_v, didx_v, rows_v, acc_sh, sem):
        cid = lax.axis_index("c")
        sid = lax.axis_index("s")
        base = (cid * NS + sid) * e_per_w
        row0 = sid * stripe
        _zero_2d(rows_v.at[0], _CHUNK, D)

        def zbody(r, c):
            pltpu.sync_copy(rows_v.at[0], acc_sh.at[pl.ds(row0 + r * _CHUNK, _CHUNK)])
            return c

        lax.fori_loop(0, stripe // _CHUNK, zbody, 0)
        plsc.subcore_barrier()

        def fetch(i, buf):
            e0 = base + i * _CHUNK
            pltpu.sync_copy(src_hbm.at[pl.ds(e0, _CHUNK)], sidx_v.at[buf])
            pltpu.sync_copy(dst_hbm.at[pl.ds(e0, _CHUNK)], didx_v.at[buf])
            return pltpu.async_copy(xs_hbm.at[sidx_v.at[buf]], rows_v.at[buf], sem)

        fetch(0, 0)

        def body(i, c):
            nxt = (i + 1) % 2

            @pl.when(i + 1 < n_chunks)
            def _():
                fetch(i + 1, nxt)

            cur = i % 2
            pltpu.make_async_copy(xs_hbm.at[sidx_v.at[cur]], rows_v.at[cur], sem).wait()
            pltpu.sync_copy(rows_v.at[cur], acc_sh.at[didx_v.at[cur]], add=True)
            return c

        lax.fori_loop(0, n_chunks, body, 0)
        plsc.subcore_barrier()
        pltpu.sync_copy(
            acc_sh.at[pl.ds(row0, stripe)],
            out_hbm.at[cid, pl.ds(row0, stripe)],
        )

    return msg_kernel


def _scale_body(deg_ref, x_ref, xs_ref, dinv_ref):
    dinv = lax.rsqrt(deg_ref[0] + deg_ref[1] + 1.0)
    xs_ref[...] = x_ref[...] * dinv
    dinv_ref[...] = dinv


def _make_scale_kernel(N, D, block_rows):
    return pl.pallas_call(
        _scale_body,
        grid=(N // block_rows,),
        in_specs=[
            pl.BlockSpec((2, block_rows, 1), lambda i: (0, i, 0)),
            pl.BlockSpec((block_rows, D), lambda i: (i, 0)),
        ],
        out_specs=[
            pl.BlockSpec((block_rows, D), lambda i: (i, 0)),
            pl.BlockSpec((block_rows, 1), lambda i: (i, 0)),
        ],
        out_shape=[
            jax.ShapeDtypeStruct((N, D), jnp.float32),
            jax.ShapeDtypeStruct((N, 1), jnp.float32),
        ],
    )


def _final_body(acc_ref, xs_ref, dinv_ref, w_ref, b_ref, out_ref):
    h = (acc_ref[0] + acc_ref[1] + xs_ref[...]) * dinv_ref[...]
    out_ref[...] = (
        jnp.dot(h, w_ref[...], preferred_element_type=jnp.float32) + b_ref[...]
    )


def _make_final_kernel(N, D, block_rows):
    return pl.pallas_call(
        _final_body,
        grid=(N // block_rows,),
        in_specs=[
            pl.BlockSpec((2, block_rows, D), lambda i: (0, i, 0)),
            pl.BlockSpec((block_rows, D), lambda i: (i, 0)),
            pl.BlockSpec((block_rows, 1), lambda i: (i, 0)),
            pl.BlockSpec((D, D), lambda i: (0, 0)),
            pl.BlockSpec((1, D), lambda i: (0, 0)),
        ],
        out_specs=pl.BlockSpec((block_rows, D), lambda i: (i, 0)),
        out_shape=jax.ShapeDtypeStruct((N, D), jnp.float32),
    )


def kernel(x, edge_index, W, b):
    N, D = x.shape
    E = edge_index.shape[1]
    NC, NS = _sc_dims()
    unit = NS * _CHUNK
    n_pad = ((N + unit - 1) // unit) * unit

    ei = edge_index.astype(jnp.int32)
    src = ei[0]
    dst = ei[1]

    deg_part = _make_degree_kernel(E, n_pad)(dst)
    deg3 = deg_part[:, :N].reshape(NC, N, 1)

    block_rows = 1000 if N % 1000 == 0 else 8
    xs, dinv = _make_scale_kernel(N, D, block_rows)(deg3, x)

    acc_part = _make_msg_kernel(E, n_pad, D)(xs, src, dst)
    acc = acc_part[:, :N, :]

    out = _make_final_kernel(N, D, block_rows)(acc, xs, dinv, W, b.reshape(1, D))
    return out
